# baseline (speedup 1.0000x reference)
import functools

import jax
import jax.numpy as jnp
from jax import lax
from jax.experimental import pallas as pl
from jax.experimental.pallas import tpu as pltpu
from jax.experimental.pallas import tpu_sc as plsc

EMBED = 32
BATCH = 16384

NUM_CORES = 2
NUM_SUBCORES = 16
NUM_WORKERS = NUM_CORES * NUM_SUBCORES
B_PER_W = BATCH // NUM_WORKERS
CHUNK = 128
N_CHUNKS = B_PER_W // CHUNK


@functools.partial(
    pl.kernel,
    mesh=plsc.VectorSubcoreMesh(core_axis_name="c", subcore_axis_name="s"),
    out_type=jax.ShapeDtypeStruct((BATCH, EMBED), jnp.bfloat16),
    scratch_types=[
        pltpu.VMEM((B_PER_W,), jnp.int32),
        pltpu.VMEM((B_PER_W, EMBED), jnp.bfloat16),
        pltpu.SemaphoreType.DMA,
    ],
    compiler_params=pltpu.CompilerParams(use_tc_tiling_on_sc=False),
)
def _gather_one(idx_hbm, table_hbm, out_hbm, idx_v, rows_v, sem):
    wid = lax.axis_index("s") * NUM_CORES + lax.axis_index("c")
    base = wid * B_PER_W

    pltpu.sync_copy(idx_hbm.at[pl.ds(base, B_PER_W)], idx_v)

    copies = []
    for j in range(N_CHUNKS):
        sl = pl.ds(j * CHUNK, CHUNK)
        copies.append(
            pltpu.async_copy(table_hbm.at[idx_v.at[sl]], rows_v.at[sl], sem)
        )
    for c in copies:
        c.wait()
    pltpu.sync_copy(rows_v, out_hbm.at[pl.ds(base, B_PER_W)])


def kernel(user, item, user_table, item_table):
    u16 = user_table.astype(jnp.bfloat16)
    i16 = item_table.astype(jnp.bfloat16)
    return (
        _gather_one(user, u16).astype(jnp.float32),
        _gather_one(item, i16).astype(jnp.float32),
    )


# speedup vs baseline: 1.1497x; 1.1497x over previous
import functools

import jax
import jax.numpy as jnp
from jax import lax
from jax.experimental import pallas as pl
from jax.experimental.pallas import tpu as pltpu
from jax.experimental.pallas import tpu_sc as plsc

EMBED = 32
BATCH = 16384

NUM_CORES = 2
NUM_SUBCORES = 16
NUM_WORKERS = NUM_CORES * NUM_SUBCORES
B_PER_W = BATCH // NUM_WORKERS
CHUNK = 128
N_CHUNKS = B_PER_W // CHUNK
L = 16


@functools.partial(
    pl.kernel,
    mesh=plsc.VectorSubcoreMesh(core_axis_name="c", subcore_axis_name="s"),
    out_type=jax.ShapeDtypeStruct((BATCH, EMBED), jnp.float32),
    scratch_types=[
        pltpu.VMEM((B_PER_W,), jnp.int32),
        pltpu.VMEM((B_PER_W,), jnp.int32),
        pltpu.VMEM((CHUNK, 128), jnp.float32),
        pltpu.VMEM((CHUNK, 128), jnp.float32),
        pltpu.VMEM((B_PER_W, EMBED), jnp.float32),
        pltpu.SemaphoreType.DMA,
        pltpu.SemaphoreType.DMA,
    ],
    compiler_params=pltpu.CompilerParams(
        use_tc_tiling_on_sc=True, needs_layout_passes=False
    ),
)
def _gather_packed(
    idx_hbm,
    ptable_hbm,
    out_hbm,
    idx_v,
    pidx_v,
    packed_a,
    packed_b,
    rows_v,
    sem_a,
    sem_b,
):
    wid = lax.axis_index("s") * NUM_CORES + lax.axis_index("c")
    base = wid * B_PER_W

    pltpu.sync_copy(idx_hbm.at[pl.ds(base, B_PER_W)], idx_v)

    def shift_body(k, _):
        sl = pl.ds(k * L, L)
        pidx_v[sl] = lax.shift_right_logical(idx_v[sl], 2)
        return 0

    lax.fori_loop(0, B_PER_W // L, shift_body, 0)

    bufs = (packed_a, packed_b)
    sems = (sem_a, sem_b)

    def start(j):
        return pltpu.async_copy(
            ptable_hbm.at[pidx_v.at[pl.ds(j * CHUNK, CHUNK)]],
            bufs[j % 2],
            sems[j % 2],
        )

    iota = jax.lax.iota(jnp.int32, L)

    def extract(j):
        buf = bufs[j % 2]

        def grp_body(g, _):
            rvec = g * L + iota
            lane0 = (idx_v[pl.ds(j * CHUNK + g * L, L)] & 3) * EMBED

            def col_body(c, _):
                vals = plsc.load_gather(buf, [rvec, lane0 + c])
                plsc.store_scatter(
                    rows_v,
                    [j * CHUNK + rvec, jnp.broadcast_to(c, (L,))],
                    vals,
                )
                return 0

            lax.fori_loop(0, EMBED, col_body, 0)
            return 0

        lax.fori_loop(0, CHUNK // L, grp_body, 0)

    pending = start(0)
    for j in range(N_CHUNKS):
        nxt = start(j + 1) if j + 1 < N_CHUNKS else None
        pending.wait()
        extract(j)
        pending = nxt

    pltpu.sync_copy(rows_v, out_hbm.at[pl.ds(base, B_PER_W)])


def kernel(user, item, user_table, item_table):
    up = user_table.reshape(250000, 128)
    ip = item_table.reshape(250000, 128)
    return (_gather_packed(user, up), _gather_packed(item, ip))


# speedup vs baseline: 1.1591x; 1.0082x over previous
import functools

import jax
import jax.numpy as jnp
from jax import lax
from jax.experimental import pallas as pl
from jax.experimental.pallas import tpu as pltpu
from jax.experimental.pallas import tpu_sc as plsc

EMBED = 32
BATCH = 16384

NUM_CORES = 2
NUM_SUBCORES = 16
NUM_WORKERS = NUM_CORES * NUM_SUBCORES
B_PER_W = BATCH // NUM_WORKERS
CHUNK = 128
N_CHUNKS = B_PER_W // CHUNK


@functools.partial(
    pl.kernel,
    mesh=plsc.VectorSubcoreMesh(core_axis_name="c", subcore_axis_name="s"),
    out_type=jax.ShapeDtypeStruct((BATCH, EMBED), jnp.float32),
    scratch_types=[
        pltpu.VMEM((B_PER_W,), jnp.int32),
        pltpu.VMEM((B_PER_W, EMBED), jnp.float32),
        pltpu.SemaphoreType.DMA,
    ],
    compiler_params=pltpu.CompilerParams(use_tc_tiling_on_sc=False),
)
def _gather_one(idx_hbm, table_hbm, out_hbm, idx_v, rows_v, sem):
    wid = lax.axis_index("s") * NUM_CORES + lax.axis_index("c")
    base = wid * B_PER_W

    pltpu.sync_copy(idx_hbm.at[pl.ds(base, B_PER_W)], idx_v)

    copies = []
    for j in range(N_CHUNKS):
        sl = pl.ds(j * CHUNK, CHUNK)
        copies.append(
            pltpu.async_copy(table_hbm.at[idx_v.at[sl]], rows_v.at[sl], sem)
        )
    for c in copies:
        c.wait()
    pltpu.sync_copy(rows_v, out_hbm.at[pl.ds(base, B_PER_W)])


def kernel(user, item, user_table, item_table):
    return (_gather_one(user, user_table), _gather_one(item, item_table))
